# Initial kernel scaffold; baseline (speedup 1.0000x reference)
import jax
import jax.numpy as jnp
from jax.experimental import pallas as pl


def kernel(feature, edge_index, W, b, gamma, beta):
    raise NotImplementedError("write your pallas kernel here")


# speedup vs baseline: 4.9492x; 4.9492x over previous
import functools

import jax
import jax.numpy as jnp
from jax import lax
from jax.experimental import pallas as pl
from jax.experimental.pallas import tpu as pltpu
from jax.experimental.pallas import tpu_sc as plsc

N_NODES = 10000
FEATS = 128
N_EDGES = 320000
EPS = 1e-5

NC = 2
NS = 16
NW = NC * NS
CHUNK = 128
CHUNKS = (N_EDGES // NW + CHUNK - 1) // CHUNK
E_PAD = NW * CHUNKS * CHUNK
ROWS = 10240
RPT = ROWS // NS
DUMMY = N_NODES


def _leaky_relu_tc(x):
    def body(x_ref, o_ref):
        v = x_ref[...]
        o_ref[...] = jnp.where(v > 0, v, jnp.float32(0.2) * v)

    return pl.pallas_call(
        body,
        out_shape=jax.ShapeDtypeStruct(x.shape, x.dtype),
    )(x)


def _sc_segment_sum(h, src3, dst3, zrows):
    mesh = plsc.VectorSubcoreMesh(core_axis_name="c", subcore_axis_name="s")

    @functools.partial(
        pl.kernel,
        mesh=mesh,
        out_type=jax.ShapeDtypeStruct((NC, ROWS, FEATS), jnp.float32),
        scratch_types=[
            pltpu.VMEM((CHUNKS, CHUNK), jnp.int32),
            pltpu.VMEM((CHUNKS, CHUNK), jnp.int32),
            pltpu.VMEM((CHUNK, FEATS), jnp.float32),
            pltpu.VMEM_SHARED((ROWS, FEATS), jnp.float32),
            pltpu.SemaphoreType.DMA,
        ],
    )
    def k(h_hbm, src_hbm, dst_hbm, z_hbm, out_hbm, src_v, dst_v, rows_v, acc, sem):
        c = lax.axis_index("c")
        s = lax.axis_index("s")
        wid = s * NC + c
        pltpu.sync_copy(z_hbm, acc.at[pl.ds(s * RPT, RPT)])
        pltpu.sync_copy(src_hbm.at[wid], src_v)
        pltpu.sync_copy(dst_hbm.at[wid], dst_v)
        plsc.subcore_barrier()

        def body(j, carry):
            pltpu.async_copy(h_hbm.at[src_v.at[j]], rows_v, sem).wait()
            pltpu.sync_copy(rows_v, acc.at[dst_v.at[j]], add=True)
            return carry

        lax.fori_loop(0, CHUNKS, body, 0)
        plsc.subcore_barrier()
        pltpu.sync_copy(acc.at[pl.ds(s * RPT, RPT)],
                        out_hbm.at[c, pl.ds(s * RPT, RPT)])

    return k(h, src3, dst3, zrows)


def _tc_finish(p0, p1, wt, b2, g2, be2):
    def body(p0_ref, p1_ref, wt_ref, b_ref, g_ref, be_ref, o_ref):
        agg = p0_ref[...] + p1_ref[...]
        h2 = jnp.dot(agg, wt_ref[...], preferred_element_type=jnp.float32)
        h2 = h2 + b_ref[...]
        mean = jnp.mean(h2, axis=0, keepdims=True)
        ctr = h2 - mean
        var = jnp.mean(ctr * ctr, axis=0, keepdims=True)
        o_ref[...] = g_ref[...] * ctr * lax.rsqrt(var + EPS) + be_ref[...]

    return pl.pallas_call(
        body,
        out_shape=jax.ShapeDtypeStruct((N_NODES, FEATS), jnp.float32),
    )(p0, p1, wt, b2, g2, be2)


def kernel(feature, edge_index, W, b, gamma, beta):
    h = _leaky_relu_tc(feature)
    ei = edge_index.astype(jnp.int32)
    pad = E_PAD - N_EDGES
    src_p = jnp.concatenate(
        [ei[0], jnp.zeros((pad,), jnp.int32)]).reshape(NW, CHUNKS, CHUNK)
    dst_p = jnp.concatenate(
        [ei[1], jnp.full((pad,), DUMMY, jnp.int32)]).reshape(NW, CHUNKS, CHUNK)
    zrows = jnp.zeros((RPT, FEATS), jnp.float32)
    parts = _sc_segment_sum(h, src_p, dst_p, zrows)
    p0 = parts[0, :N_NODES]
    p1 = parts[1, :N_NODES]
    return _tc_finish(p0, p1, W.T,
                      b.reshape(1, FEATS),
                      gamma.reshape(1, FEATS),
                      beta.reshape(1, FEATS))
